# baseline (speedup 1.0000x reference)
import functools

import jax
import jax.numpy as jnp
from jax import lax
from jax.experimental import pallas as pl
from jax.experimental.pallas import tpu as pltpu
from jax.experimental.pallas import tpu_sc as plsc

R_MAX = 5.0
NUM_POINTS = 5000
NUM_BASIS = 8
INV_H = (NUM_POINTS - 1) / R_MAX
LANES = 16
BLK = 128
STRIDE = 9
G_PAD = -(-NUM_POINTS * STRIDE // LANES) * LANES


def _sc_gaussian(x, w16, r_values, g_flat, *, num_cores, num_subcores):
    n = x.shape[0]
    nw = num_cores * num_subcores
    chunk = 3200
    n_chunks = n // chunk
    outer = -(-n_chunks // nw)
    assert n_chunks * chunk == n and chunk % BLK == 0 and outer % 2 == 0

    mesh = plsc.VectorSubcoreMesh(
        core_axis_name="c",
        subcore_axis_name="s",
        num_cores=num_cores,
        num_subcores=num_subcores,
    )

    @functools.partial(
        pl.kernel,
        out_type=jax.ShapeDtypeStruct((n // BLK, NUM_BASIS, BLK), jnp.float32),
        mesh=mesh,
        compiler_params=pltpu.CompilerParams(
            needs_layout_passes=False, use_tc_tiling_on_sc=False
        ),
        scratch_types=[
            pltpu.VMEM((NUM_POINTS,), jnp.float32),
            pltpu.VMEM((G_PAD,), jnp.float32),
            pltpu.VMEM((LANES,), jnp.float32),
            pltpu.VMEM((2, chunk), jnp.float32),
            pltpu.VMEM((2, chunk // BLK, NUM_BASIS, BLK), jnp.float32),
            pltpu.SemaphoreType.DMA,
            pltpu.SemaphoreType.DMA,
            pltpu.SemaphoreType.DMA,
            pltpu.SemaphoreType.DMA,
        ],
    )
    def run(
        x_hbm, w_hbm, r_hbm, g_hbm, out_hbm,
        r_v, g_v, w_v, xb, ob, si0, si1, so0, so1,
    ):
        wid = lax.axis_index("s") * num_cores + lax.axis_index("c")
        pltpu.sync_copy(r_hbm, r_v)
        pltpu.sync_copy(g_hbm, g_v)
        pltpu.sync_copy(w_hbm, w_v)
        sin = (si0, si1)
        sout = (so0, so1)

        lane = lax.iota(jnp.int32, LANES)

        @plsc.parallel_loop(0, G_PAD, step=LANES, unroll=4)
        def scale_body(i):
            col = (i + lane) % STRIDE
            wv = plsc.load_gather(w_v, [col])
            g_v[pl.ds(i, LANES)] = g_v[pl.ds(i, LANES)] * wv

        def valid(t):
            return wid + t * nw < n_chunks

        def start_in(t, b):
            cid = wid + t * nw
            pltpu.async_copy(
                x_hbm.at[pl.ds(cid * chunk, chunk)], xb.at[b], sin[b]
            )

        nblk = chunk // BLK
        h0 = nblk // 2
        h1 = nblk - h0

        def start_out(t, b):
            cid = wid + t * nw
            obase = cid * nblk
            pltpu.async_copy(
                ob.at[b, pl.ds(0, h0)],
                out_hbm.at[pl.ds(obase, h0)],
                sout[b],
            )
            pltpu.async_copy(
                ob.at[b, pl.ds(h0, h1)],
                out_hbm.at[pl.ds(obase + h0, h1)],
                sout[b],
            )

        def wait_in(b):
            pltpu.make_async_copy(
                x_hbm.at[pl.ds(0, chunk)], xb.at[b], sin[b]
            ).wait()

        def wait_out(b):
            pltpu.make_async_copy(
                ob.at[b, pl.ds(0, h0)], out_hbm.at[pl.ds(0, h0)], sout[b]
            ).wait()
            pltpu.make_async_copy(
                ob.at[b, pl.ds(h0, h1)], out_hbm.at[pl.ds(0, h1)], sout[b]
            ).wait()

        def compute(b):
            @plsc.parallel_loop(0, chunk // BLK, unroll=5)
            def blk_body(blk):
                for g in range(BLK // LANES):
                    xv = xb[b, pl.ds(blk * BLK + g * LANES, LANES)]
                    xc = jnp.minimum(xv, R_MAX)
                    k = (xc * INV_H).astype(jnp.int32)
                    r0 = plsc.load_gather(r_v, [k])
                    idx9 = (k + (r0 < xc).astype(jnp.int32)) * STRIDE
                    for j in range(NUM_BASIS):
                        gv = plsc.load_gather(g_v, [idx9 + j])
                        ob[b, blk, j, pl.ds(g * LANES, LANES)] = gv

        @pl.when(valid(0))
        def _prologue():
            start_in(0, 0)

        def pair_body(tt, carry):
            for b in range(2):
                t = tt * 2 + b

                @pl.when(valid(t + 1))
                def _prefetch():
                    start_in(t + 1, 1 - b)

                @pl.when(jnp.logical_and(valid(t), t >= 2))
                def _drain_out():
                    wait_out(b)

                @pl.when(valid(t))
                def _work():
                    wait_in(b)
                    compute(b)
                    start_out(t, b)

            return carry

        lax.fori_loop(0, outer // 2, pair_body, 0, unroll=False)

        @pl.when(valid(outer - 2))
        def _tail0():
            wait_out((outer - 2) % 2)

        @pl.when(valid(outer - 1))
        def _tail1():
            wait_out((outer - 1) % 2)

    return run(x, w16, r_values, g_flat)


def kernel(x, gaussian_weights, r_values, gaussian_values):
    info = plsc.get_sparse_core_info()
    n = x.shape[0]
    w16 = jnp.pad(gaussian_weights, (0, LANES - NUM_BASIS))
    g9 = jnp.pad(
        gaussian_values, ((0, 0), (0, STRIDE - NUM_BASIS))
    ).reshape(-1)
    g9 = jnp.pad(g9, (0, G_PAD - g9.shape[0]))
    out = _sc_gaussian(
        x,
        w16,
        r_values,
        g9,
        num_cores=info.num_cores,
        num_subcores=info.num_subcores,
    )
    return out.transpose(0, 2, 1).reshape(n, NUM_BASIS)


# speedup vs baseline: 1.2663x; 1.2663x over previous
import functools

import jax
import jax.numpy as jnp
from jax import lax
from jax.experimental import pallas as pl
from jax.experimental.pallas import tpu as pltpu
from jax.experimental.pallas import tpu_sc as plsc

R_MAX = 5.0
NUM_POINTS = 5000
NUM_BASIS = 8
INV_H = (NUM_POINTS - 1) / R_MAX
LANES = 16
BLK = 128
STRIDE = 9
G_PAD = -(-NUM_POINTS * STRIDE // LANES) * LANES


def _sc_gaussian(x, w16, r_values, g_flat, *, num_cores, num_subcores):
    n = x.shape[0]
    nw = num_cores * num_subcores
    chunk = 3200
    n_chunks = n // chunk
    outer = -(-n_chunks // nw)
    assert n_chunks * chunk == n and chunk % BLK == 0 and outer % 2 == 0

    mesh = plsc.VectorSubcoreMesh(
        core_axis_name="c",
        subcore_axis_name="s",
        num_cores=num_cores,
        num_subcores=num_subcores,
    )

    @functools.partial(
        pl.kernel,
        out_type=jax.ShapeDtypeStruct((n // BLK, NUM_BASIS, BLK), jnp.float32),
        mesh=mesh,
        compiler_params=pltpu.CompilerParams(
            needs_layout_passes=False, use_tc_tiling_on_sc=False
        ),
        scratch_types=[
            pltpu.VMEM((NUM_POINTS,), jnp.float32),
            pltpu.VMEM((G_PAD,), jnp.float32),
            pltpu.VMEM((LANES,), jnp.float32),
            pltpu.VMEM((2, chunk), jnp.float32),
            pltpu.VMEM((2, chunk // BLK, NUM_BASIS, BLK), jnp.float32),
            pltpu.SemaphoreType.DMA,
            pltpu.SemaphoreType.DMA,
            pltpu.SemaphoreType.DMA,
            pltpu.SemaphoreType.DMA,
        ],
    )
    def run(
        x_hbm, w_hbm, r_hbm, g_hbm, out_hbm,
        r_v, g_v, w_v, xb, ob, si0, si1, so0, so1,
    ):
        wid = lax.axis_index("s") * num_cores + lax.axis_index("c")
        pltpu.sync_copy(r_hbm, r_v)
        pltpu.sync_copy(g_hbm, g_v)
        pltpu.sync_copy(w_hbm, w_v)
        sin = (si0, si1)
        sout = (so0, so1)

        lane = lax.iota(jnp.int32, LANES)

        @plsc.parallel_loop(0, G_PAD, step=LANES, unroll=4)
        def scale_body(i):
            col = (i + lane) % STRIDE
            wv = plsc.load_gather(w_v, [col])
            g_v[pl.ds(i, LANES)] = g_v[pl.ds(i, LANES)] * wv

        def valid(t):
            return wid + t * nw < n_chunks

        def start_in(t, b):
            cid = wid + t * nw
            pltpu.async_copy(
                x_hbm.at[pl.ds(cid * chunk, chunk)], xb.at[b], sin[b]
            )

        nblk = chunk // BLK
        h0 = nblk // 2
        h1 = nblk - h0

        def start_out(t, b):
            cid = wid + t * nw
            obase = cid * nblk
            pltpu.async_copy(
                ob.at[b, pl.ds(0, h0)],
                out_hbm.at[pl.ds(obase, h0)],
                sout[b],
            )
            pltpu.async_copy(
                ob.at[b, pl.ds(h0, h1)],
                out_hbm.at[pl.ds(obase + h0, h1)],
                sout[b],
            )

        def wait_in(b):
            pltpu.make_async_copy(
                x_hbm.at[pl.ds(0, chunk)], xb.at[b], sin[b]
            ).wait()

        def wait_out(b):
            pltpu.make_async_copy(
                ob.at[b, pl.ds(0, h0)], out_hbm.at[pl.ds(0, h0)], sout[b]
            ).wait()
            pltpu.make_async_copy(
                ob.at[b, pl.ds(h0, h1)], out_hbm.at[pl.ds(0, h1)], sout[b]
            ).wait()

        def compute(b):
            @plsc.parallel_loop(0, chunk // BLK, unroll=3)
            def blk_body(blk):
                for g in range(BLK // LANES):
                    xv = xb[b, pl.ds(blk * BLK + g * LANES, LANES)]
                    xc = jnp.minimum(xv, R_MAX)
                    k = (xc * INV_H).astype(jnp.int32)
                    r0 = plsc.load_gather(r_v, [k])
                    idx9 = (k + (r0 < xc).astype(jnp.int32)) * STRIDE
                    for j in range(NUM_BASIS):
                        gv = plsc.load_gather(g_v, [idx9 + j])
                        ob[b, blk, j, pl.ds(g * LANES, LANES)] = gv

        @pl.when(valid(0))
        def _prologue():
            start_in(0, 0)

        def pair_body(tt, carry):
            for b in range(2):
                t = tt * 2 + b

                @pl.when(valid(t + 1))
                def _prefetch():
                    start_in(t + 1, 1 - b)

                @pl.when(jnp.logical_and(valid(t), t >= 2))
                def _drain_out():
                    wait_out(b)

                @pl.when(valid(t))
                def _work():
                    wait_in(b)
                    compute(b)
                    start_out(t, b)

            return carry

        lax.fori_loop(0, outer // 2, pair_body, 0, unroll=False)

        @pl.when(valid(outer - 2))
        def _tail0():
            wait_out((outer - 2) % 2)

        @pl.when(valid(outer - 1))
        def _tail1():
            wait_out((outer - 1) % 2)

    return run(x, w16, r_values, g_flat)


def kernel(x, gaussian_weights, r_values, gaussian_values):
    info = plsc.get_sparse_core_info()
    n = x.shape[0]
    w16 = jnp.pad(gaussian_weights, (0, LANES - NUM_BASIS))
    g9 = jnp.pad(
        gaussian_values, ((0, 0), (0, STRIDE - NUM_BASIS))
    ).reshape(-1)
    g9 = jnp.pad(g9, (0, G_PAD - g9.shape[0]))
    out = _sc_gaussian(
        x,
        w16,
        r_values,
        g9,
        num_cores=info.num_cores,
        num_subcores=info.num_subcores,
    )
    return out.transpose(0, 2, 1).reshape(n, NUM_BASIS)
